# Initial kernel scaffold; baseline (speedup 1.0000x reference)
import jax
import jax.numpy as jnp
from jax.experimental import pallas as pl


def kernel(pt_enc, ig_enc, cg_enc, gg_enc, item_num, pt_tab, ig_tab, cg_tab, gg_tab, ln0_g, ln0_b, W1, b1, ln1_g, ln1_b, W2, b2):
    raise NotImplementedError("write your pallas kernel here")


# speedup vs baseline: 1.0190x; 1.0190x over previous
import functools

import jax
import jax.numpy as jnp
from jax import lax
from jax.experimental import pallas as pl
from jax.experimental.pallas import tpu as pltpu
from jax.experimental.pallas import tpu_sc as plsc

B = 16384
EMB = 32
HID = 256
OUT = 128
NUM = 3
N_FEAT = 131

_NC, _NS = 2, 16
_NW = _NC * _NS
_BPW = B // _NW


def _sc_gather_body(pt_i, ig_i, cg_i, gg_i, pt_t, ig_t, cg_t, gg_t, out,
                    idx0, idx1, idx2, idx3, r0, r1, r2, r3, sem):
    wid = lax.axis_index("s") * _NC + lax.axis_index("c")
    base = wid * _BPW
    idxs = (idx0, idx1, idx2, idx3)
    rows = (r0, r1, r2, r3)
    tabs = (pt_t, ig_t, cg_t, gg_t)
    encs = (pt_i, ig_i, cg_i, gg_i)
    for t in range(4):
        pltpu.sync_copy(encs[t].at[pl.ds(base, _BPW)], idxs[t])
    cps = [pltpu.async_copy(tabs[t].at[idxs[t]], rows[t], sem)
           for t in range(4)]
    for cp in cps:
        cp.wait()
    for t in range(4):
        pltpu.sync_copy(rows[t], out.at[pl.ds(base, _BPW),
                                        pl.ds(t * EMB, EMB)])


@functools.cache
def _sc_gather():
    return pl.kernel(
        _sc_gather_body,
        out_type=jax.ShapeDtypeStruct((B, 4 * EMB), jnp.float32),
        mesh=plsc.VectorSubcoreMesh(core_axis_name="c", subcore_axis_name="s",
                                    num_cores=_NC, num_subcores=_NS),
        compiler_params=pltpu.CompilerParams(use_tc_tiling_on_sc=False),
        scratch_types=(
            [pltpu.VMEM((_BPW,), jnp.int32) for _ in range(4)]
            + [pltpu.VMEM((_BPW, EMB), jnp.float32) for _ in range(4)]
            + [pltpu.SemaphoreType.DMA]
        ),
    )


_BBLK = 1024


def _tc_mlp_body(e_ref, num_ref, ge_ref, be_ref, gn_ref, bn_ref,
                 w1a_ref, w1b_ref, b1_ref, g1_ref, bb1_ref,
                 w2_ref, b2_ref, o_ref):
    e = e_ref[...]
    num = num_ref[...]
    inv_n = 1.0 / N_FEAT
    s = jnp.sum(e, axis=-1, keepdims=True) + jnp.sum(num, axis=-1, keepdims=True)
    mu = s * inv_n
    ss = (jnp.sum(e * e, axis=-1, keepdims=True)
          + jnp.sum(num * num, axis=-1, keepdims=True))
    var = ss * inv_n - mu * mu
    rstd = lax.rsqrt(var + 1e-5)
    en = (e - mu) * rstd * ge_ref[...] + be_ref[...]
    nn = (num - mu) * rstd * gn_ref[...] + bn_ref[...]
    h = (jnp.dot(en, w1a_ref[...], preferred_element_type=jnp.float32)
         + jnp.dot(nn, w1b_ref[...], preferred_element_type=jnp.float32)
         + b1_ref[...])
    h = jnp.maximum(h, 0.0)
    mu1 = jnp.mean(h, axis=-1, keepdims=True)
    var1 = jnp.mean(h * h, axis=-1, keepdims=True) - mu1 * mu1
    hn = (h - mu1) * lax.rsqrt(var1 + 1e-5) * g1_ref[...] + bb1_ref[...]
    o = jnp.dot(hn, w2_ref[...], preferred_element_type=jnp.float32) + b2_ref[...]
    nrm = jnp.maximum(jnp.sqrt(jnp.sum(o * o, axis=-1, keepdims=True)), 1e-8)
    o_ref[...] = o / nrm


def _full(shape):
    return pl.BlockSpec(shape, lambda i: (0,) * len(shape))


_tc_mlp = pl.pallas_call(
    _tc_mlp_body,
    grid=(B // _BBLK,),
    in_specs=[
        pl.BlockSpec((_BBLK, 4 * EMB), lambda i: (i, 0)),
        pl.BlockSpec((_BBLK, NUM), lambda i: (i, 0)),
        _full((1, 4 * EMB)), _full((1, 4 * EMB)),
        _full((1, NUM)), _full((1, NUM)),
        _full((4 * EMB, HID)), _full((NUM, HID)), _full((1, HID)),
        _full((1, HID)), _full((1, HID)),
        _full((HID, OUT)), _full((1, OUT)),
    ],
    out_specs=pl.BlockSpec((_BBLK, OUT), lambda i: (i, 0)),
    out_shape=jax.ShapeDtypeStruct((B, OUT), jnp.float32),
)


@jax.jit
def kernel(pt_enc, ig_enc, cg_enc, gg_enc, item_num, pt_tab, ig_tab, cg_tab,
           gg_tab, ln0_g, ln0_b, W1, b1, ln1_g, ln1_b, W2, b2):
    e = _sc_gather()(pt_enc.astype(jnp.int32), ig_enc.astype(jnp.int32),
                   cg_enc.astype(jnp.int32), gg_enc.astype(jnp.int32),
                   pt_tab, ig_tab, cg_tab, gg_tab)
    ge = ln0_g[:4 * EMB].reshape(1, -1)
    be = ln0_b[:4 * EMB].reshape(1, -1)
    gn = ln0_g[4 * EMB:].reshape(1, -1)
    bn = ln0_b[4 * EMB:].reshape(1, -1)
    return _tc_mlp(e, item_num, ge, be, gn, bn,
                   W1[:4 * EMB], W1[4 * EMB:], b1.reshape(1, -1),
                   ln1_g.reshape(1, -1), ln1_b.reshape(1, -1),
                   W2, b2.reshape(1, -1))
